# baseline (speedup 1.0000x reference)
import functools

import jax
import jax.numpy as jnp
from jax import lax
from jax.experimental import pallas as pl
from jax.experimental.pallas import tpu as pltpu
from jax.experimental.pallas import tpu_sc as plsc

B = 128
N = 32768
K = 1024
L = 16
NBUCKET = 2048
CAP = 2048
M = CAP // L
NC = 2
NS = 16
NW = NC * NS
ROWS_PER = B // NW


def _body(x_hbm, sel_hbm, perm_hbm, row_v, hist_v, ckK_v, ckV_v, tkK_v,
          tkV_v, outp_v):
    wid = lax.axis_index("s") * NC + lax.axis_index("c")
    lane = lax.iota(jnp.int32, L)
    ones = jnp.ones((L,), jnp.int32)
    zeros16 = jnp.zeros((L,), jnp.int32)
    neginf = jnp.full((L,), -jnp.inf, jnp.float32)

    def do_row(t, _row_carry):
        r = wid * ROWS_PER + t
        with jax.named_scope("p0_dma_in"):
            pltpu.sync_copy(x_hbm.at[pl.ds(r * N, N)], row_v)

        def zb(i, c):
            hist_v[pl.ds(i * L, L)] = zeros16
            return c

        with jax.named_scope("p1_zero"):
            lax.fori_loop(0, NBUCKET, zb, 0, unroll=8)

        def hb(i, c):
            v = row_v[pl.ds(i * L, L)]
            b = lax.bitcast_convert_type(v, jnp.int32)
            m = jnp.where(b < 0, b ^ jnp.int32(0x7FFFFFFF), b)
            bk = (m >> 21) + 1024
            plsc.addupdate_scatter(hist_v, [bk * L + lane], ones)
            return c

        with jax.named_scope("p2_hist"):
            lax.fori_loop(0, N // L, hb, 0, unroll=8)

        def blk_cond(c):
            _g, _cum, done = c
            return jnp.logical_not(done)

        def blk_body(c):
            g, cum, _done = c

            def acc(q, s):
                return s + hist_v[pl.ds(g * (16 * L) + q * L, L)]

            s = lax.fori_loop(0, 16, acc, zeros16, unroll=16)
            bc = jnp.sum(s)
            d = cum + bc >= K
            return (jnp.where(d, g, g - 1), jnp.where(d, cum, cum + bc), d)

        with jax.named_scope("p3_scan_a"):
            g, cum0, _ = lax.while_loop(
                blk_cond, blk_body,
                (jnp.int32(NBUCKET // 16 - 1), jnp.int32(0),
                 jnp.bool_(False)))

        def fb_cond(c):
            _b, _cum, done = c
            return jnp.logical_not(done)

        def fb_body(c):
            bkt, cum, _done = c
            cb = jnp.sum(hist_v[pl.ds(bkt * L, L)])
            d = cum + cb >= K
            return (jnp.where(d, bkt, bkt - 1), jnp.where(d, cum, cum + cb),
                    d)

        with jax.named_scope("p3_scan_b"):
            T, _A, _ = lax.while_loop(fb_cond, fb_body,
                                      (g * 16 + 15, cum0, jnp.bool_(False)))

        def pf(i, c):
            ckK_v[pl.ds(i * L, L)] = neginf
            return c

        with jax.named_scope("p4_prefill"):
            lax.fori_loop(0, M + 1, pf, 0, unroll=8)

        def cb_(i, off):
            v = row_v[pl.ds(i * L, L)]
            b = lax.bitcast_convert_type(v, jnp.int32)
            m = jnp.where(b < 0, b ^ jnp.int32(0x7FFFFFFF), b)
            bk = (m >> 21) + 1024
            mask = bk >= T
            cnt = plsc.all_reduce_population_count(mask)[0]

            @pl.when(off <= CAP)
            def _():
                plsc.store_compressed(ckK_v.at[pl.ds(off, L)], v, mask=mask)
                plsc.store_compressed(ckV_v.at[pl.ds(off, L)], i * L + lane,
                                      mask=mask)

            return jnp.minimum(off + cnt, jnp.int32(CAP))

        with jax.named_scope("p5_compact"):
            lax.fori_loop(0, N // L, cb_, jnp.int32(0), unroll=4)

        def ps(i, c):
            k = ckK_v[pl.ds(i * L, L)]
            v = ckV_v[pl.ds(i * L, L)]
            k2, v2 = plsc.sort_key_val(k, v, descending=True)
            ckK_v[pl.ds(i * L, L)] = k2
            ckV_v[pl.ds(i * L, L)] = v2
            return c

        with jax.named_scope("p6_presort"):
            lax.fori_loop(0, M, ps, 0, unroll=4)

        def merge_level(srcK, srcV, dstK, dstV, w):
            last = 2 * w == M

            def pair(p, c):
                s = p * (2 * w)
                e1 = s + w
                e2 = s + 2 * w

                def copy_run():
                    def cp(q, cc):
                        dstK[pl.ds((s + q) * L, L)] = srcK[pl.ds((s + q) * L,
                                                                 L)]
                        dstV[pl.ds((s + q) * L, L)] = srcV[pl.ds((s + q) * L,
                                                                 L)]
                        return cc

                    lax.fori_loop(0, w, cp, 0, unroll=4)
                    if not last:
                        def cp2(q, cc):
                            dstK[pl.ds((e1 + q) * L, L)] = neginf
                            dstV[pl.ds((e1 + q) * L, L)] = zeros16
                            return cc

                        lax.fori_loop(0, w, cp2, 0, unroll=4)

                def merge_run():
                    tA0 = (srcK[pl.ds(s * L, L)][0] >=
                           srcK[pl.ds(e1 * L, L)][0])
                    first = jnp.where(tA0, s, e1)
                    vK0 = srcK[pl.ds(first * L, L)]
                    vV0 = srcV[pl.ds(first * L, L)]
                    i0 = jnp.where(tA0, s + 1, s)
                    j0 = jnp.where(tA0, e1, e1 + 1)

                    def step(o, carry):
                        i, j, vK, vV = carry
                        canA = i < e1
                        canB = j < e2
                        headA = srcK[pl.ds(i * L, L)][0]
                        headB = srcK[pl.ds(j * L, L)][0]
                        tA = canA & (jnp.logical_not(canB) | (headA >= headB))
                        tt = jnp.where(tA, i, j)
                        uK = srcK[pl.ds(tt * L, L)]
                        uV = srcV[pl.ds(tt * L, L)]
                        i2 = jnp.where(tA, i + 1, i)
                        j2 = jnp.where(tA, j, j + 1)
                        ruK = lax.rev(uK, (0,))
                        ruV = lax.rev(uV, (0,))
                        m2 = vK >= ruK
                        hiK = jnp.where(m2, vK, ruK)
                        hiV = jnp.where(m2, vV, ruV)
                        loK = jnp.where(m2, ruK, vK)
                        loV = jnp.where(m2, ruV, vV)
                        hiK, hiV = plsc.sort_key_val(hiK, hiV,
                                                     descending=True)
                        loK, loV = plsc.sort_key_val(loK, loV,
                                                     descending=True)
                        dstK[pl.ds((s + o) * L, L)] = hiK
                        dstV[pl.ds((s + o) * L, L)] = hiV
                        return (i2, j2, loK, loV)

                    nsteps = (K // L) if last else (2 * w - 1)
                    _i, _j, vK, vV = lax.fori_loop(0, nsteps, step,
                                                   (i0, j0, vK0, vV0))
                    if not last:
                        dstK[pl.ds((e2 - 1) * L, L)] = vK
                        dstV[pl.ds((e2 - 1) * L, L)] = vV

                run2_all_pad = srcK[pl.ds(e1 * L, L)][0] == -jnp.inf
                lax.cond(run2_all_pad, copy_run, merge_run)
                return c

            lax.fori_loop(0, M // (2 * w), pair, 0)

        bufs = ((ckK_v, ckV_v), (tkK_v, tkV_v))
        src = 0
        w = 1
        while w < M:
            sK, sV = bufs[src]
            dK, dV = bufs[1 - src]
            with jax.named_scope("p7_merge_w%d" % w):
                merge_level(sK, sV, dK, dV, w)
            src = 1 - src
            w *= 2
        finK, finV = bufs[src]

        def ob(i, c):
            ivec = finV[pl.ds(i * L, L)]
            outp_v[pl.ds(i * L, L)] = ivec + r * N
            return c

        with jax.named_scope("p8_out"):
            lax.fori_loop(0, K // L, ob, 0)
            pltpu.sync_copy(finK.at[pl.ds(0, K)], sel_hbm.at[r])
            pltpu.sync_copy(outp_v, perm_hbm.at[pl.ds(r * K, K)])
        return _row_carry

    lax.fori_loop(0, ROWS_PER, do_row, 0)


@functools.partial(jax.jit, static_argnames=())
def _topk_sc(x):
    mesh = plsc.VectorSubcoreMesh(core_axis_name="c", subcore_axis_name="s")
    fn = pl.kernel(
        _body,
        mesh=mesh,
        compiler_params=pltpu.CompilerParams(needs_layout_passes=False),
        out_type=(
            jax.ShapeDtypeStruct((B, K), jnp.float32),
            jax.ShapeDtypeStruct((B * K,), jnp.int32),
        ),
        scratch_types=[
            pltpu.VMEM((N,), jnp.float32),
            pltpu.VMEM((NBUCKET * L,), jnp.int32),
            pltpu.VMEM((CAP + L,), jnp.float32),
            pltpu.VMEM((CAP + L,), jnp.int32),
            pltpu.VMEM((CAP + L,), jnp.float32),
            pltpu.VMEM((CAP + L,), jnp.int32),
            pltpu.VMEM((K,), jnp.int32),
        ],
    )
    return fn(x)


def kernel(x, num_nodes):
    sel, perm = _topk_sc(x)
    ks = jnp.full((B,), K, dtype=num_nodes.dtype)
    return sel, perm, ks


# speedup vs baseline: 1.0873x; 1.0873x over previous
import functools

import jax
import jax.numpy as jnp
from jax import lax
from jax.experimental import pallas as pl
from jax.experimental.pallas import tpu as pltpu
from jax.experimental.pallas import tpu_sc as plsc

B = 128
N = 32768
K = 1024
L = 16
NBUCKET = 2048
CAP = 2048
M = CAP // L
NC = 2
NS = 16
NW = NC * NS
ROWS_PER = B // NW


def _body(x_hbm, sel_hbm, perm_hbm, row_v, hist_v, ckK_v, ckV_v, tkK_v,
          tkV_v, outp_v):
    wid = lax.axis_index("s") * NC + lax.axis_index("c")
    lane = lax.iota(jnp.int32, L)
    ones = jnp.ones((L,), jnp.int32)
    zeros16 = jnp.zeros((L,), jnp.int32)
    neginf = jnp.full((L,), -jnp.inf, jnp.float32)

    def zb(i, c):
        hist_v[pl.ds(i * L, L)] = zeros16
        return c

    lax.fori_loop(0, NBUCKET, zb, 0, unroll=8)

    def do_row(t, _row_carry):
        r = wid * ROWS_PER + t
        with jax.named_scope("p0_dma_in"):
            pltpu.sync_copy(x_hbm.at[pl.ds(r * N, N)], row_v)


        def hb(i, c):
            v = row_v[pl.ds(i * L, L)]
            b = lax.bitcast_convert_type(v, jnp.int32)
            m = jnp.where(b < 0, b ^ jnp.int32(0x7FFFFFFF), b)
            bk = (m >> 21) + 1024
            plsc.addupdate_scatter(hist_v, [bk * L + lane], ones)
            return c

        with jax.named_scope("p2_hist"):
            lax.fori_loop(0, N // L, hb, 0, unroll=8)

        def blk_cond(c):
            _g, _cum, done = c
            return jnp.logical_not(done)

        def blk_body(c):
            g, cum, _done = c

            def acc(q, s):
                return s + hist_v[pl.ds(g * (16 * L) + q * L, L)]

            s = lax.fori_loop(0, 16, acc, zeros16, unroll=16)
            bc = jnp.sum(s)
            d = cum + bc >= K
            return (jnp.where(d, g, g - 1), jnp.where(d, cum, cum + bc), d)

        with jax.named_scope("p3_scan_a"):
            g, cum0, _ = lax.while_loop(
                blk_cond, blk_body,
                (jnp.int32(NBUCKET // 16 - 1), jnp.int32(0),
                 jnp.bool_(False)))

        def fb_cond(c):
            _b, _cum, done = c
            return jnp.logical_not(done)

        def fb_body(c):
            bkt, cum, _done = c
            cb = jnp.sum(hist_v[pl.ds(bkt * L, L)])
            d = cum + cb >= K
            return (jnp.where(d, bkt, bkt - 1), jnp.where(d, cum, cum + cb),
                    d)

        with jax.named_scope("p3_scan_b"):
            T, _A, _ = lax.while_loop(fb_cond, fb_body,
                                      (g * 16 + 15, cum0, jnp.bool_(False)))

        def pf(i, c):
            ckK_v[pl.ds(i * L, L)] = neginf
            return c

        with jax.named_scope("p4_prefill"):
            lax.fori_loop(0, M + 1, pf, 0, unroll=8)

        def cb_(i, off):
            v = row_v[pl.ds(i * L, L)]
            b = lax.bitcast_convert_type(v, jnp.int32)
            m = jnp.where(b < 0, b ^ jnp.int32(0x7FFFFFFF), b)
            bk = (m >> 21) + 1024
            mask = bk >= T
            cnt = plsc.all_reduce_population_count(mask)[0]
            soff = jnp.minimum(off, jnp.int32(CAP))
            plsc.store_compressed(ckK_v.at[pl.ds(soff, L)], v, mask=mask)
            plsc.store_compressed(ckV_v.at[pl.ds(soff, L)], i * L + lane,
                                  mask=mask)
            hist_v[pl.ds(i * L, L)] = zeros16
            return off + cnt

        with jax.named_scope("p5_compact"):
            lax.fori_loop(0, N // L, cb_, jnp.int32(0), unroll=8)

        def ps(i, c):
            k = ckK_v[pl.ds(i * L, L)]
            v = ckV_v[pl.ds(i * L, L)]
            k2, v2 = plsc.sort_key_val(k, v, descending=True)
            ckK_v[pl.ds(i * L, L)] = k2
            ckV_v[pl.ds(i * L, L)] = v2
            return c

        with jax.named_scope("p6_presort"):
            lax.fori_loop(0, M, ps, 0, unroll=4)

        def merge_level(srcK, srcV, dstK, dstV, w):
            last = 2 * w == M

            def pair(p, c):
                s = p * (2 * w)
                e1 = s + w
                e2 = s + 2 * w

                def copy_run():
                    def cp(q, cc):
                        dstK[pl.ds((s + q) * L, L)] = srcK[pl.ds((s + q) * L,
                                                                 L)]
                        dstV[pl.ds((s + q) * L, L)] = srcV[pl.ds((s + q) * L,
                                                                 L)]
                        return cc

                    lax.fori_loop(0, w, cp, 0, unroll=4)
                    if not last:
                        def cp2(q, cc):
                            dstK[pl.ds((e1 + q) * L, L)] = neginf
                            dstV[pl.ds((e1 + q) * L, L)] = zeros16
                            return cc

                        lax.fori_loop(0, w, cp2, 0, unroll=4)

                def merge_run():
                    aK0 = srcK[pl.ds(s * L, L)]
                    bK0 = srcK[pl.ds(e1 * L, L)]
                    tA0 = aK0[0] >= bK0[0]
                    vK0 = jnp.where(tA0, aK0, bK0)
                    aV0 = srcV[pl.ds(s * L, L)]
                    bV0 = srcV[pl.ds(e1 * L, L)]
                    vV0 = jnp.where(tA0, aV0, bV0)
                    i0 = jnp.where(tA0, s + 1, s)
                    j0 = jnp.where(tA0, e1, e1 + 1)
                    hA0 = srcK[pl.ds(i0 * L, L)][0]
                    hB0 = srcK[pl.ds(j0 * L, L)][0]

                    def step(o, carry):
                        i, j, hA, hB, vK, vV = carry
                        canA = i < e1
                        canB = j < e2
                        tA = canA & (jnp.logical_not(canB) | (hA >= hB))
                        tt = jnp.where(tA, i, j)
                        uK = srcK[pl.ds(tt * L, L)]
                        uV = srcV[pl.ds(tt * L, L)]
                        i2 = jnp.where(tA, i + 1, i)
                        j2 = jnp.where(tA, j, j + 1)
                        nxt = jnp.where(tA, i2, j2)
                        nh = srcK[pl.ds(nxt * L, L)][0]
                        hA2 = jnp.where(tA, nh, hA)
                        hB2 = jnp.where(tA, hB, nh)
                        ruK = lax.rev(uK, (0,))
                        ruV = lax.rev(uV, (0,))
                        m2 = vK >= ruK
                        hiK = jnp.where(m2, vK, ruK)
                        hiV = jnp.where(m2, vV, ruV)
                        loK = jnp.where(m2, ruK, vK)
                        loV = jnp.where(m2, ruV, vV)
                        hiK, hiV = plsc.sort_key_val(hiK, hiV,
                                                     descending=True)
                        loK, loV = plsc.sort_key_val(loK, loV,
                                                     descending=True)
                        dstK[pl.ds((s + o) * L, L)] = hiK
                        dstV[pl.ds((s + o) * L, L)] = hiV
                        return (i2, j2, hA2, hB2, loK, loV)

                    nsteps = (K // L) if last else (2 * w - 1)
                    out = lax.fori_loop(0, nsteps, step,
                                        (i0, j0, hA0, hB0, vK0, vV0))
                    if not last:
                        vK, vV = out[4], out[5]
                        dstK[pl.ds((e2 - 1) * L, L)] = vK
                        dstV[pl.ds((e2 - 1) * L, L)] = vV

                run2_all_pad = srcK[pl.ds(e1 * L, L)][0] == -jnp.inf
                lax.cond(run2_all_pad, copy_run, merge_run)
                return c

            lax.fori_loop(0, M // (2 * w), pair, 0)

        bufs = ((ckK_v, ckV_v), (tkK_v, tkV_v))
        src = 0
        w = 1
        while w < M:
            sK, sV = bufs[src]
            dK, dV = bufs[1 - src]
            with jax.named_scope("p7_merge_w%d" % w):
                merge_level(sK, sV, dK, dV, w)
            src = 1 - src
            w *= 2
        finK, finV = bufs[src]

        def ob(i, c):
            ivec = finV[pl.ds(i * L, L)]
            outp_v[pl.ds(i * L, L)] = ivec + r * N
            return c

        with jax.named_scope("p8_out"):
            lax.fori_loop(0, K // L, ob, 0)
            pltpu.sync_copy(finK.at[pl.ds(0, K)], sel_hbm.at[r])
            pltpu.sync_copy(outp_v, perm_hbm.at[pl.ds(r * K, K)])
        return _row_carry

    lax.fori_loop(0, ROWS_PER, do_row, 0)


@functools.partial(jax.jit, static_argnames=())
def _topk_sc(x):
    mesh = plsc.VectorSubcoreMesh(core_axis_name="c", subcore_axis_name="s")
    fn = pl.kernel(
        _body,
        mesh=mesh,
        compiler_params=pltpu.CompilerParams(needs_layout_passes=False),
        out_type=(
            jax.ShapeDtypeStruct((B, K), jnp.float32),
            jax.ShapeDtypeStruct((B * K,), jnp.int32),
        ),
        scratch_types=[
            pltpu.VMEM((N,), jnp.float32),
            pltpu.VMEM((NBUCKET * L,), jnp.int32),
            pltpu.VMEM((CAP + L,), jnp.float32),
            pltpu.VMEM((CAP + L,), jnp.int32),
            pltpu.VMEM((CAP + L,), jnp.float32),
            pltpu.VMEM((CAP + L,), jnp.int32),
            pltpu.VMEM((K,), jnp.int32),
        ],
    )
    return fn(x)


def kernel(x, num_nodes):
    sel, perm = _topk_sc(x)
    ks = jnp.full((B,), K, dtype=num_nodes.dtype)
    return sel, perm, ks


# speedup vs baseline: 1.1123x; 1.0230x over previous
import functools

import jax
import jax.numpy as jnp
from jax import lax
from jax.experimental import pallas as pl
from jax.experimental.pallas import tpu as pltpu
from jax.experimental.pallas import tpu_sc as plsc

B = 128
N = 32768
K = 1024
L = 16
NBUCKET = 2048
CAP = 2048
M = CAP // L
NC = 2
NS = 16
NW = NC * NS
ROWS_PER = B // NW


def _body(x_hbm, sel_hbm, perm_hbm, row_v, hist_v, ckK_v, ckV_v, tkK_v,
          tkV_v, outp_v, dma_sem):
    wid = lax.axis_index("s") * NC + lax.axis_index("c")
    lane = lax.iota(jnp.int32, L)
    ones = jnp.ones((L,), jnp.int32)
    zeros16 = jnp.zeros((L,), jnp.int32)
    neginf = jnp.full((L,), -jnp.inf, jnp.float32)

    def zb(i, c):
        hist_v[pl.ds(i * L, L)] = zeros16
        return c

    pltpu.async_copy(x_hbm.at[pl.ds(wid * ROWS_PER * N, N)], row_v, dma_sem)
    lax.fori_loop(0, NBUCKET, zb, 0, unroll=8)

    def do_row(t, _row_carry):
        r = wid * ROWS_PER + t
        with jax.named_scope("p0_dma_wait"):
            pltpu.make_async_copy(x_hbm.at[pl.ds(r * N, N)], row_v,
                                  dma_sem).wait()


        def hb(i, c):
            v = row_v[pl.ds(i * L, L)]
            b = lax.bitcast_convert_type(v, jnp.int32)
            m = jnp.where(b < 0, b ^ jnp.int32(0x7FFFFFFF), b)
            bk = (m >> 21) + 1024
            plsc.addupdate_scatter(hist_v, [bk * L + lane], ones)
            return c

        with jax.named_scope("p2_hist"):
            lax.fori_loop(0, N // L, hb, 0, unroll=8)

        def blk_cond(c):
            _g, _cum, done = c
            return jnp.logical_not(done)

        def blk_body(c):
            g, cum, _done = c

            def acc(q, s):
                return s + hist_v[pl.ds(g * (16 * L) + q * L, L)]

            s = lax.fori_loop(0, 16, acc, zeros16, unroll=16)
            bc = jnp.sum(s)
            d = cum + bc >= K
            return (jnp.where(d, g, g - 1), jnp.where(d, cum, cum + bc), d)

        with jax.named_scope("p3_scan_a"):
            g, cum0, _ = lax.while_loop(
                blk_cond, blk_body,
                (jnp.int32(NBUCKET // 16 - 1), jnp.int32(0),
                 jnp.bool_(False)))

        def fb_cond(c):
            _b, _cum, done = c
            return jnp.logical_not(done)

        def fb_body(c):
            bkt, cum, _done = c
            cb = jnp.sum(hist_v[pl.ds(bkt * L, L)])
            d = cum + cb >= K
            return (jnp.where(d, bkt, bkt - 1), jnp.where(d, cum, cum + cb),
                    d)

        with jax.named_scope("p3_scan_b"):
            T, _A, _ = lax.while_loop(fb_cond, fb_body,
                                      (g * 16 + 15, cum0, jnp.bool_(False)))

        def pf(i, c):
            ckK_v[pl.ds(i * L, L)] = neginf
            return c

        with jax.named_scope("p4_prefill"):
            lax.fori_loop(0, M + 1, pf, 0, unroll=8)

        def cb_(i, off):
            v = row_v[pl.ds(i * L, L)]
            b = lax.bitcast_convert_type(v, jnp.int32)
            m = jnp.where(b < 0, b ^ jnp.int32(0x7FFFFFFF), b)
            bk = (m >> 21) + 1024
            mask = bk >= T
            cnt = plsc.all_reduce_population_count(mask)[0]
            soff = jnp.minimum(off, jnp.int32(CAP))
            plsc.store_compressed(ckK_v.at[pl.ds(soff, L)], v, mask=mask)
            plsc.store_compressed(ckV_v.at[pl.ds(soff, L)], i * L + lane,
                                  mask=mask)
            hist_v[pl.ds(i * L, L)] = zeros16
            return off + cnt

        with jax.named_scope("p5_compact"):
            lax.fori_loop(0, N // L, cb_, jnp.int32(0), unroll=8)

        @pl.when(t < ROWS_PER - 1)
        def _prefetch():
            pltpu.async_copy(x_hbm.at[pl.ds((r + 1) * N, N)], row_v, dma_sem)

        def ps(i, c):
            k = ckK_v[pl.ds(i * L, L)]
            v = ckV_v[pl.ds(i * L, L)]
            k2, v2 = plsc.sort_key_val(k, v, descending=True)
            ckK_v[pl.ds(i * L, L)] = k2
            ckV_v[pl.ds(i * L, L)] = v2
            return c

        with jax.named_scope("p6_presort"):
            lax.fori_loop(0, M, ps, 0, unroll=8)

        def merge_level(srcK, srcV, dstK, dstV, w):
            last = 2 * w == M

            def pair(p, c):
                s = p * (2 * w)
                e1 = s + w
                e2 = s + 2 * w

                def copy_run():
                    def cp(q, cc):
                        dstK[pl.ds((s + q) * L, L)] = srcK[pl.ds((s + q) * L,
                                                                 L)]
                        dstV[pl.ds((s + q) * L, L)] = srcV[pl.ds((s + q) * L,
                                                                 L)]
                        return cc

                    lax.fori_loop(0, w, cp, 0, unroll=4)
                    if not last:
                        def cp2(q, cc):
                            dstK[pl.ds((e1 + q) * L, L)] = neginf
                            dstV[pl.ds((e1 + q) * L, L)] = zeros16
                            return cc

                        lax.fori_loop(0, w, cp2, 0, unroll=4)

                def merge_run():
                    aK0 = srcK[pl.ds(s * L, L)]
                    bK0 = srcK[pl.ds(e1 * L, L)]
                    tA0 = aK0[0] >= bK0[0]
                    vK0 = jnp.where(tA0, aK0, bK0)
                    aV0 = srcV[pl.ds(s * L, L)]
                    bV0 = srcV[pl.ds(e1 * L, L)]
                    vV0 = jnp.where(tA0, aV0, bV0)
                    i0 = jnp.where(tA0, s + 1, s)
                    j0 = jnp.where(tA0, e1, e1 + 1)
                    hA0 = srcK[pl.ds(i0 * L, L)][0]
                    hB0 = srcK[pl.ds(j0 * L, L)][0]

                    def step(o, carry):
                        i, j, hA, hB, vK, vV = carry
                        canA = i < e1
                        canB = j < e2
                        tA = canA & (jnp.logical_not(canB) | (hA >= hB))
                        tt = jnp.where(tA, i, j)
                        uK = srcK[pl.ds(tt * L, L)]
                        uV = srcV[pl.ds(tt * L, L)]
                        i2 = jnp.where(tA, i + 1, i)
                        j2 = jnp.where(tA, j, j + 1)
                        nxt = jnp.where(tA, i2, j2)
                        nh = srcK[pl.ds(nxt * L, L)][0]
                        hA2 = jnp.where(tA, nh, hA)
                        hB2 = jnp.where(tA, hB, nh)
                        ruK = lax.rev(uK, (0,))
                        ruV = lax.rev(uV, (0,))
                        m2 = vK >= ruK
                        hiK = jnp.where(m2, vK, ruK)
                        hiV = jnp.where(m2, vV, ruV)
                        loK = jnp.where(m2, ruK, vK)
                        loV = jnp.where(m2, ruV, vV)
                        loK, loV = plsc.sort_key_val(loK, loV,
                                                     descending=True)
                        hiK, hiV = plsc.sort_key_val(hiK, hiV,
                                                     descending=True)
                        dstK[pl.ds((s + o) * L, L)] = hiK
                        dstV[pl.ds((s + o) * L, L)] = hiV
                        return (i2, j2, hA2, hB2, loK, loV)

                    nsteps = (K // L) if last else (2 * w - 1)
                    out = lax.fori_loop(0, nsteps, step,
                                        (i0, j0, hA0, hB0, vK0, vV0))
                    if not last:
                        vK, vV = out[4], out[5]
                        dstK[pl.ds((e2 - 1) * L, L)] = vK
                        dstV[pl.ds((e2 - 1) * L, L)] = vV

                run2_all_pad = srcK[pl.ds(e1 * L, L)][0] == -jnp.inf
                lax.cond(run2_all_pad, copy_run, merge_run)
                return c

            lax.fori_loop(0, M // (2 * w), pair, 0)

        bufs = ((ckK_v, ckV_v), (tkK_v, tkV_v))
        src = 0
        w = 1
        while w < M:
            sK, sV = bufs[src]
            dK, dV = bufs[1 - src]
            with jax.named_scope("p7_merge_w%d" % w):
                merge_level(sK, sV, dK, dV, w)
            src = 1 - src
            w *= 2
        finK, finV = bufs[src]

        def ob(i, c):
            ivec = finV[pl.ds(i * L, L)]
            outp_v[pl.ds(i * L, L)] = ivec + r * N
            return c

        with jax.named_scope("p8_out"):
            lax.fori_loop(0, K // L, ob, 0)
            pltpu.sync_copy(finK.at[pl.ds(0, K)], sel_hbm.at[r])
            pltpu.sync_copy(outp_v, perm_hbm.at[pl.ds(r * K, K)])
        return _row_carry

    lax.fori_loop(0, ROWS_PER, do_row, 0)


@functools.partial(jax.jit, static_argnames=())
def _topk_sc(x):
    mesh = plsc.VectorSubcoreMesh(core_axis_name="c", subcore_axis_name="s")
    fn = pl.kernel(
        _body,
        mesh=mesh,
        compiler_params=pltpu.CompilerParams(needs_layout_passes=False),
        out_type=(
            jax.ShapeDtypeStruct((B, K), jnp.float32),
            jax.ShapeDtypeStruct((B * K,), jnp.int32),
        ),
        scratch_types=[
            pltpu.VMEM((N,), jnp.float32),
            pltpu.VMEM((NBUCKET * L,), jnp.int32),
            pltpu.VMEM((CAP + L,), jnp.float32),
            pltpu.VMEM((CAP + L,), jnp.int32),
            pltpu.VMEM((CAP + L,), jnp.float32),
            pltpu.VMEM((CAP + L,), jnp.int32),
            pltpu.VMEM((K,), jnp.int32),
            pltpu.SemaphoreType.DMA,
        ],
    )
    return fn(x)


def kernel(x, num_nodes):
    sel, perm = _topk_sc(x)
    ks = jnp.full((B,), K, dtype=num_nodes.dtype)
    return sel, perm, ks


# speedup vs baseline: 2.3325x; 2.0970x over previous
import functools

import jax
import jax.numpy as jnp
from jax import lax
from jax.experimental import pallas as pl
from jax.experimental.pallas import tpu as pltpu
from jax.experimental.pallas import tpu_sc as plsc

B = 128
N = 32768
K = 1024
L = 16
NBUCKET = 2048
CAP = 2048
M = CAP // L
NC = 2
NS = 16
NW = NC * NS
ROWS_PER = B // NW


def _body(x_hbm, sel_hbm, perm_hbm, row_v, hist_v, ckK_v, ckV_v, tkK_v,
          tkV_v, outp_v, dma_sem):
    wid = lax.axis_index("s") * NC + lax.axis_index("c")
    lane = lax.iota(jnp.int32, L)
    ones = jnp.ones((L,), jnp.int32)
    zeros16 = jnp.zeros((L,), jnp.int32)
    neginf = jnp.full((L,), -jnp.inf, jnp.float32)

    pltpu.async_copy(x_hbm.at[pl.ds(wid * ROWS_PER * N, N)], row_v, dma_sem)

    @plsc.parallel_loop(0, NBUCKET, unroll=8)
    def _zb(i):
        hist_v[pl.ds(i * L, L)] = zeros16

    def do_row(t, _row_carry):
        r = wid * ROWS_PER + t
        with jax.named_scope("p0_dma_wait"):
            pltpu.make_async_copy(x_hbm.at[pl.ds(r * N, N)], row_v,
                                  dma_sem).wait()


        with jax.named_scope("p2_hist"):
            @plsc.parallel_loop(0, N // L, unroll=8)
            def _hb(i):
                v = row_v[pl.ds(i * L, L)]
                b = lax.bitcast_convert_type(v, jnp.int32)
                m = jnp.where(b < 0, b ^ jnp.int32(0x7FFFFFFF), b)
                bk = (m >> 21) + 1024
                plsc.addupdate_scatter(hist_v, [bk * L + lane], ones)

        def blk_cond(c):
            _g, _cum, done = c
            return jnp.logical_not(done)

        def blk_body(c):
            g, cum, _done = c

            def acc(q, s):
                return s + hist_v[pl.ds(g * (16 * L) + q * L, L)]

            s = lax.fori_loop(0, 16, acc, zeros16, unroll=16)
            bc = jnp.sum(s)
            d = cum + bc >= K
            return (jnp.where(d, g, g - 1), jnp.where(d, cum, cum + bc), d)

        with jax.named_scope("p3_scan_a"):
            g, cum0, _ = lax.while_loop(
                blk_cond, blk_body,
                (jnp.int32(NBUCKET // 16 - 1), jnp.int32(0),
                 jnp.bool_(False)))

        def fb_cond(c):
            _b, _cum, done = c
            return jnp.logical_not(done)

        def fb_body(c):
            bkt, cum, _done = c
            cb = jnp.sum(hist_v[pl.ds(bkt * L, L)])
            d = cum + cb >= K
            return (jnp.where(d, bkt, bkt - 1), jnp.where(d, cum, cum + cb),
                    d)

        with jax.named_scope("p3_scan_b"):
            T, _A, _ = lax.while_loop(fb_cond, fb_body,
                                      (g * 16 + 15, cum0, jnp.bool_(False)))

        with jax.named_scope("p4_prefill"):
            @plsc.parallel_loop(0, M + 1, unroll=8)
            def _pf(i):
                ckK_v[pl.ds(i * L, L)] = neginf

        with jax.named_scope("p5_compact"):
            @plsc.parallel_loop(0, N // L, unroll=8, carry=jnp.int32(0))
            def _cb(i, off):
                v = row_v[pl.ds(i * L, L)]
                b = lax.bitcast_convert_type(v, jnp.int32)
                m = jnp.where(b < 0, b ^ jnp.int32(0x7FFFFFFF), b)
                bk = (m >> 21) + 1024
                mask = bk >= T
                cnt = plsc.all_reduce_population_count(mask)[0]
                soff = jnp.minimum(off, jnp.int32(CAP))
                plsc.store_compressed(ckK_v.at[pl.ds(soff, L)], v, mask=mask)
                plsc.store_compressed(ckV_v.at[pl.ds(soff, L)],
                                      i * L + lane, mask=mask)
                hist_v[pl.ds(i * L, L)] = zeros16
                return off + cnt

        @pl.when(t < ROWS_PER - 1)
        def _prefetch():
            pltpu.async_copy(x_hbm.at[pl.ds((r + 1) * N, N)], row_v, dma_sem)

        with jax.named_scope("p6_presort"):
            @plsc.parallel_loop(0, M, unroll=8)
            def _ps(i):
                k = ckK_v[pl.ds(i * L, L)]
                v = ckV_v[pl.ds(i * L, L)]
                k2, v2 = plsc.sort_key_val(k, v, descending=True)
                ckK_v[pl.ds(i * L, L)] = k2
                ckV_v[pl.ds(i * L, L)] = v2

        def merge_level(srcK, srcV, dstK, dstV, w):
            last = 2 * w == M

            def pair(p, c):
                s = p * (2 * w)
                e1 = s + w
                e2 = s + 2 * w

                def copy_run():
                    def cp(q, cc):
                        dstK[pl.ds((s + q) * L, L)] = srcK[pl.ds((s + q) * L,
                                                                 L)]
                        dstV[pl.ds((s + q) * L, L)] = srcV[pl.ds((s + q) * L,
                                                                 L)]
                        return cc

                    lax.fori_loop(0, w, cp, 0, unroll=4)
                    if not last:
                        def cp2(q, cc):
                            dstK[pl.ds((e1 + q) * L, L)] = neginf
                            dstV[pl.ds((e1 + q) * L, L)] = zeros16
                            return cc

                        lax.fori_loop(0, w, cp2, 0, unroll=4)

                def merge_run():
                    aK0 = srcK[pl.ds(s * L, L)]
                    bK0 = srcK[pl.ds(e1 * L, L)]
                    tA0 = aK0[0] >= bK0[0]
                    vK0 = jnp.where(tA0, aK0, bK0)
                    aV0 = srcV[pl.ds(s * L, L)]
                    bV0 = srcV[pl.ds(e1 * L, L)]
                    vV0 = jnp.where(tA0, aV0, bV0)
                    i0 = jnp.where(tA0, s + 1, s)
                    j0 = jnp.where(tA0, e1, e1 + 1)
                    hA0 = srcK[pl.ds(i0 * L, L)][0]
                    hB0 = srcK[pl.ds(j0 * L, L)][0]

                    def step(o, carry):
                        i, j, hA, hB, vK, vV = carry
                        canA = i < e1
                        canB = j < e2
                        tA = canA & (jnp.logical_not(canB) | (hA >= hB))
                        tt = jnp.where(tA, i, j)
                        uK = srcK[pl.ds(tt * L, L)]
                        uV = srcV[pl.ds(tt * L, L)]
                        i2 = jnp.where(tA, i + 1, i)
                        j2 = jnp.where(tA, j, j + 1)
                        nxt = jnp.where(tA, i2, j2)
                        nh = srcK[pl.ds(nxt * L, L)][0]
                        hA2 = jnp.where(tA, nh, hA)
                        hB2 = jnp.where(tA, hB, nh)
                        ruK = lax.rev(uK, (0,))
                        ruV = lax.rev(uV, (0,))
                        m2 = vK >= ruK
                        hiK = jnp.where(m2, vK, ruK)
                        hiV = jnp.where(m2, vV, ruV)
                        loK = jnp.where(m2, ruK, vK)
                        loV = jnp.where(m2, ruV, vV)
                        loK, loV = plsc.sort_key_val(loK, loV,
                                                     descending=True)
                        hiK, hiV = plsc.sort_key_val(hiK, hiV,
                                                     descending=True)
                        dstK[pl.ds((s + o) * L, L)] = hiK
                        dstV[pl.ds((s + o) * L, L)] = hiV
                        return (i2, j2, hA2, hB2, loK, loV)

                    nsteps = (K // L) if last else (2 * w - 1)
                    out = lax.fori_loop(0, nsteps, step,
                                        (i0, j0, hA0, hB0, vK0, vV0))
                    if not last:
                        vK, vV = out[4], out[5]
                        dstK[pl.ds((e2 - 1) * L, L)] = vK
                        dstV[pl.ds((e2 - 1) * L, L)] = vV

                run2_all_pad = srcK[pl.ds(e1 * L, L)][0] == -jnp.inf
                lax.cond(run2_all_pad, copy_run, merge_run)
                return c

            lax.fori_loop(0, M // (2 * w), pair, 0)

        bufs = ((ckK_v, ckV_v), (tkK_v, tkV_v))
        src = 0
        w = 1
        while w < M:
            sK, sV = bufs[src]
            dK, dV = bufs[1 - src]
            with jax.named_scope("p7_merge_w%d" % w):
                merge_level(sK, sV, dK, dV, w)
            src = 1 - src
            w *= 2
        finK, finV = bufs[src]

        def ob(i, c):
            ivec = finV[pl.ds(i * L, L)]
            outp_v[pl.ds(i * L, L)] = ivec + r * N
            return c

        with jax.named_scope("p8_out"):
            lax.fori_loop(0, K // L, ob, 0)
            pltpu.sync_copy(finK.at[pl.ds(0, K)], sel_hbm.at[r])
            pltpu.sync_copy(outp_v, perm_hbm.at[pl.ds(r * K, K)])
        return _row_carry

    lax.fori_loop(0, ROWS_PER, do_row, 0)


@functools.partial(jax.jit, static_argnames=())
def _topk_sc(x):
    mesh = plsc.VectorSubcoreMesh(core_axis_name="c", subcore_axis_name="s")
    fn = pl.kernel(
        _body,
        mesh=mesh,
        compiler_params=pltpu.CompilerParams(needs_layout_passes=False),
        out_type=(
            jax.ShapeDtypeStruct((B, K), jnp.float32),
            jax.ShapeDtypeStruct((B * K,), jnp.int32),
        ),
        scratch_types=[
            pltpu.VMEM((N,), jnp.float32),
            pltpu.VMEM((NBUCKET * L,), jnp.int32),
            pltpu.VMEM((CAP + L,), jnp.float32),
            pltpu.VMEM((CAP + L,), jnp.int32),
            pltpu.VMEM((CAP + L,), jnp.float32),
            pltpu.VMEM((CAP + L,), jnp.int32),
            pltpu.VMEM((K,), jnp.int32),
            pltpu.SemaphoreType.DMA,
        ],
    )
    return fn(x)


def kernel(x, num_nodes):
    sel, perm = _topk_sc(x)
    ks = jnp.full((B,), K, dtype=num_nodes.dtype)
    return sel, perm, ks


# speedup vs baseline: 2.5723x; 1.1028x over previous
import functools

import jax
import jax.numpy as jnp
from jax import lax
from jax.experimental import pallas as pl
from jax.experimental.pallas import tpu as pltpu
from jax.experimental.pallas import tpu_sc as plsc

B = 128
N = 32768
K = 1024
L = 16
NBUCKET = 2048
CAP = 2048
M = CAP // L
NC = 2
NS = 16
NW = NC * NS
ROWS_PER = B // NW


def _body(x_hbm, sel_hbm, perm_hbm, row_v, hist_v, ckK_v, ckV_v, tkK_v,
          tkV_v, outp_v, dma_sem):
    wid = lax.axis_index("s") * NC + lax.axis_index("c")
    lane = lax.iota(jnp.int32, L)
    ones = jnp.ones((L,), jnp.int32)
    zeros16 = jnp.zeros((L,), jnp.int32)
    neginf = jnp.full((L,), -jnp.inf, jnp.float32)

    pltpu.async_copy(x_hbm.at[pl.ds(wid * ROWS_PER * N, N)], row_v, dma_sem)

    @plsc.parallel_loop(0, NBUCKET, unroll=8)
    def _zb(i):
        hist_v[pl.ds(i * L, L)] = zeros16

    def do_row(t, _row_carry):
        r = wid * ROWS_PER + t
        with jax.named_scope("p0_dma_wait"):
            pltpu.make_async_copy(x_hbm.at[pl.ds(r * N, N)], row_v,
                                  dma_sem).wait()


        with jax.named_scope("p2_hist"):
            @plsc.parallel_loop(0, N // L, unroll=8)
            def _hb(i):
                v = row_v[pl.ds(i * L, L)]
                b = lax.bitcast_convert_type(v, jnp.int32)
                m = jnp.where(b < 0, b ^ jnp.int32(0x7FFFFFFF), b)
                bk = (m >> 21) + 1024
                plsc.addupdate_scatter(hist_v, [bk * L + lane], ones)

        def blk_cond(c):
            _g, _cum, done = c
            return jnp.logical_not(done)

        def blk_body(c):
            g, cum, _done = c

            def acc(q, s):
                return s + hist_v[pl.ds(g * (16 * L) + q * L, L)]

            s = lax.fori_loop(0, 16, acc, zeros16, unroll=16)
            bc = jnp.sum(s)
            d = cum + bc >= K
            return (jnp.where(d, g, g - 1), jnp.where(d, cum, cum + bc), d)

        with jax.named_scope("p3_scan_a"):
            g, cum0, _ = lax.while_loop(
                blk_cond, blk_body,
                (jnp.int32(NBUCKET // 16 - 1), jnp.int32(0),
                 jnp.bool_(False)))

        def fb_cond(c):
            _b, _cum, done = c
            return jnp.logical_not(done)

        def fb_body(c):
            bkt, cum, _done = c
            cb = jnp.sum(hist_v[pl.ds(bkt * L, L)])
            d = cum + cb >= K
            return (jnp.where(d, bkt, bkt - 1), jnp.where(d, cum, cum + cb),
                    d)

        with jax.named_scope("p3_scan_b"):
            T, _A, _ = lax.while_loop(fb_cond, fb_body,
                                      (g * 16 + 15, cum0, jnp.bool_(False)))

        with jax.named_scope("p4_prefill"):
            @plsc.parallel_loop(0, M + 1, unroll=8)
            def _pf(i):
                ckK_v[pl.ds(i * L, L)] = neginf

        with jax.named_scope("p5_compact"):
            @plsc.parallel_loop(0, N // L, unroll=8, carry=jnp.int32(0))
            def _cb(i, off):
                v = row_v[pl.ds(i * L, L)]
                b = lax.bitcast_convert_type(v, jnp.int32)
                m = jnp.where(b < 0, b ^ jnp.int32(0x7FFFFFFF), b)
                bk = (m >> 21) + 1024
                mask = bk >= T
                cnt = plsc.all_reduce_population_count(mask)[0]
                soff = jnp.minimum(off, jnp.int32(CAP))
                plsc.store_compressed(ckK_v.at[pl.ds(soff, L)], v, mask=mask)
                plsc.store_compressed(ckV_v.at[pl.ds(soff, L)],
                                      i * L + lane, mask=mask)
                hist_v[pl.ds(i * L, L)] = zeros16
                return off + cnt

        @pl.when(t < ROWS_PER - 1)
        def _prefetch():
            pltpu.async_copy(x_hbm.at[pl.ds((r + 1) * N, N)], row_v, dma_sem)

        with jax.named_scope("p6_presort"):
            @plsc.parallel_loop(0, M, unroll=8)
            def _ps(i):
                k = ckK_v[pl.ds(i * L, L)]
                v = ckV_v[pl.ds(i * L, L)]
                k2, v2 = plsc.sort_key_val(k, v, descending=True)
                ckK_v[pl.ds(i * L, L)] = k2
                ckV_v[pl.ds(i * L, L)] = v2

        def merge_level(srcK, srcV, dstK, dstV, w):
            last = 2 * w == M

            def pair(p):
                s = p * (2 * w)
                e1 = s + w
                e2 = s + 2 * w

                def copy_run():
                    @plsc.parallel_loop(0, w, unroll=4)
                    def _cp(q):
                        dstK[pl.ds((s + q) * L, L)] = srcK[pl.ds((s + q) * L,
                                                                 L)]
                        dstV[pl.ds((s + q) * L, L)] = srcV[pl.ds((s + q) * L,
                                                                 L)]

                    if not last:
                        @plsc.parallel_loop(0, w, unroll=4)
                        def _cp2(q):
                            dstK[pl.ds((e1 + q) * L, L)] = neginf
                            dstV[pl.ds((e1 + q) * L, L)] = zeros16

                def merge_run():
                    aK0 = srcK[pl.ds(s * L, L)]
                    bK0 = srcK[pl.ds(e1 * L, L)]
                    tA0 = aK0[0] >= bK0[0]
                    vK0 = jnp.where(tA0, aK0, bK0)
                    aV0 = srcV[pl.ds(s * L, L)]
                    bV0 = srcV[pl.ds(e1 * L, L)]
                    vV0 = jnp.where(tA0, aV0, bV0)
                    i0 = jnp.where(tA0, s + 1, s)
                    j0 = jnp.where(tA0, e1, e1 + 1)
                    hA0 = srcK[pl.ds(i0 * L, L)][0]
                    hB0 = srcK[pl.ds(j0 * L, L)][0]

                    def step(o, carry):
                        i, j, hA, hB, vK, vV = carry
                        canA = i < e1
                        canB = j < e2
                        tA = canA & (jnp.logical_not(canB) | (hA >= hB))
                        tt = jnp.where(tA, i, j)
                        uK = srcK[pl.ds(tt * L, L)]
                        uV = srcV[pl.ds(tt * L, L)]
                        i2 = jnp.where(tA, i + 1, i)
                        j2 = jnp.where(tA, j, j + 1)
                        nxt = jnp.where(tA, i2, j2)
                        nh = srcK[pl.ds(nxt * L, L)][0]
                        hA2 = jnp.where(tA, nh, hA)
                        hB2 = jnp.where(tA, hB, nh)
                        ruK = lax.rev(uK, (0,))
                        ruV = lax.rev(uV, (0,))
                        m2 = vK >= ruK
                        hiK = jnp.where(m2, vK, ruK)
                        hiV = jnp.where(m2, vV, ruV)
                        loK = jnp.where(m2, ruK, vK)
                        loV = jnp.where(m2, ruV, vV)
                        loK, loV = plsc.sort_key_val(loK, loV,
                                                     descending=True)
                        hiK, hiV = plsc.sort_key_val(hiK, hiV,
                                                     descending=True)
                        dstK[pl.ds((s + o) * L, L)] = hiK
                        dstV[pl.ds((s + o) * L, L)] = hiV
                        return (i2, j2, hA2, hB2, loK, loV)

                    nsteps = (K // L) if last else (2 * w - 1)
                    out = plsc.parallel_loop(
                        0, nsteps, unroll=2,
                        carry=(i0, j0, hA0, hB0, vK0, vV0))(step)
                    if not last:
                        vK, vV = out[4], out[5]
                        dstK[pl.ds((e2 - 1) * L, L)] = vK
                        dstV[pl.ds((e2 - 1) * L, L)] = vV

                run2_all_pad = srcK[pl.ds(e1 * L, L)][0] == -jnp.inf
                lax.cond(run2_all_pad, copy_run, merge_run)

            plsc.parallel_loop(0, M // (2 * w))(pair)

        bufs = ((ckK_v, ckV_v), (tkK_v, tkV_v))
        src = 0
        w = 1
        while w < M:
            sK, sV = bufs[src]
            dK, dV = bufs[1 - src]
            with jax.named_scope("p7_merge_w%d" % w):
                merge_level(sK, sV, dK, dV, w)
            src = 1 - src
            w *= 2
        finK, finV = bufs[src]

        with jax.named_scope("p8_out"):
            @plsc.parallel_loop(0, K // L, unroll=8)
            def _ob(i):
                ivec = finV[pl.ds(i * L, L)]
                outp_v[pl.ds(i * L, L)] = ivec + r * N
            pltpu.sync_copy(finK.at[pl.ds(0, K)], sel_hbm.at[r])
            pltpu.sync_copy(outp_v, perm_hbm.at[pl.ds(r * K, K)])
        return _row_carry

    lax.fori_loop(0, ROWS_PER, do_row, 0)


@functools.partial(jax.jit, static_argnames=())
def _topk_sc(x):
    mesh = plsc.VectorSubcoreMesh(core_axis_name="c", subcore_axis_name="s")
    fn = pl.kernel(
        _body,
        mesh=mesh,
        compiler_params=pltpu.CompilerParams(needs_layout_passes=False),
        out_type=(
            jax.ShapeDtypeStruct((B, K), jnp.float32),
            jax.ShapeDtypeStruct((B * K,), jnp.int32),
        ),
        scratch_types=[
            pltpu.VMEM((N,), jnp.float32),
            pltpu.VMEM((NBUCKET * L,), jnp.int32),
            pltpu.VMEM((CAP + L,), jnp.float32),
            pltpu.VMEM((CAP + L,), jnp.int32),
            pltpu.VMEM((CAP + L,), jnp.float32),
            pltpu.VMEM((CAP + L,), jnp.int32),
            pltpu.VMEM((K,), jnp.int32),
            pltpu.SemaphoreType.DMA,
        ],
    )
    return fn(x)


def kernel(x, num_nodes):
    sel, perm = _topk_sc(x)
    ks = jnp.full((B,), K, dtype=num_nodes.dtype)
    return sel, perm, ks
